# baseline (speedup 1.0000x reference)
import functools

import jax
import jax.numpy as jnp
from jax import lax
from jax.experimental import pallas as pl
from jax.experimental.pallas import tpu as pltpu
from jax.experimental.pallas import tpu_sc as plsc

DIM = 64
PAD = 128
LANES = 16
NUM_CORES = 2
NUM_SUBCORES = 16
NUM_WORKERS = NUM_CORES * NUM_SUBCORES
CHUNK = 64


def _sc_gather(total):
    per_w = total // NUM_WORKERS
    n_chunks = per_w // CHUNK
    groups = CHUNK // LANES

    mesh = plsc.VectorSubcoreMesh(
        core_axis_name="c", subcore_axis_name="s",
        num_cores=NUM_CORES, num_subcores=NUM_SUBCORES)

    @functools.partial(
        pl.kernel,
        out_type=jax.ShapeDtypeStruct((total,), jnp.float32),
        mesh=mesh,
        compiler_params=pltpu.CompilerParams(needs_layout_passes=False),
        scratch_types=[
            pltpu.VMEM((n_chunks, CHUNK), jnp.int32),
            pltpu.VMEM((n_chunks, CHUNK), jnp.int32),
            pltpu.VMEM((n_chunks, CHUNK), jnp.int32),
            pltpu.VMEM((n_chunks, CHUNK), jnp.int32),
            pltpu.VMEM((n_chunks, CHUNK), jnp.int32),
            pltpu.VMEM((n_chunks, CHUNK), jnp.int32),
            pltpu.VMEM((CHUNK, PAD), jnp.float32),
            pltpu.VMEM((CHUNK, PAD), jnp.float32),
            pltpu.VMEM((CHUNK, PAD), jnp.float32),
            pltpu.VMEM((CHUNK, PAD), jnp.float32),
            pltpu.VMEM((CHUNK, PAD), jnp.float32),
            pltpu.VMEM((CHUNK, PAD), jnp.float32),
            pltpu.VMEM((per_w,), jnp.float32),
            pltpu.SemaphoreType.DMA,
            pltpu.SemaphoreType.DMA,
        ],
    )
    def k(hrow_hbm, hcol_hbm, rrow_hbm, rcol_hbm, trow_hbm, tcol_hbm,
          ent_hbm, rel_hbm, out_hbm,
          hrow_v, hcol_v, rrow_v, rcol_v, trow_v, tcol_v,
          h0, r0, t0, h1, r1, t1, out_v, sem0, sem1):
        wid = lax.axis_index("s") * NUM_CORES + lax.axis_index("c")
        row0 = wid * n_chunks
        pltpu.sync_copy(hrow_hbm.at[pl.ds(row0, n_chunks)], hrow_v)
        pltpu.sync_copy(hcol_hbm.at[pl.ds(row0, n_chunks)], hcol_v)
        pltpu.sync_copy(rrow_hbm.at[pl.ds(row0, n_chunks)], rrow_v)
        pltpu.sync_copy(rcol_hbm.at[pl.ds(row0, n_chunks)], rcol_v)
        pltpu.sync_copy(trow_hbm.at[pl.ds(row0, n_chunks)], trow_v)
        pltpu.sync_copy(tcol_hbm.at[pl.ds(row0, n_chunks)], tcol_v)

        lane = jnp.arange(LANES, dtype=jnp.int32)
        bufs = ((h0, r0, t0, sem0), (h1, r1, t1, sem1))

        def issue(j, buf):
            h_b, r_b, t_b, sem = buf
            pltpu.async_copy(ent_hbm.at[hrow_v.at[j]], h_b, sem)
            pltpu.async_copy(rel_hbm.at[rrow_v.at[j]], r_b, sem)
            pltpu.async_copy(ent_hbm.at[trow_v.at[j]], t_b, sem)

        def drain_compute(j, buf):
            h_b, r_b, t_b, sem = buf
            pltpu.make_async_copy(ent_hbm.at[hrow_v.at[j]], h_b, sem).wait()
            pltpu.make_async_copy(rel_hbm.at[rrow_v.at[j]], r_b, sem).wait()
            pltpu.make_async_copy(ent_hbm.at[trow_v.at[j]], t_b, sem).wait()

            for g in range(groups):
                rows = g * LANES + lane
                hc = hcol_v[j, pl.ds(g * LANES, LANES)]
                rc = rcol_v[j, pl.ds(g * LANES, LANES)]
                tc = tcol_v[j, pl.ds(g * LANES, LANES)]

                def d_body(d, acc, rows=rows, hc=hc, rc=rc, tc=tc):
                    col = jnp.full((LANES,), d, dtype=jnp.int32)
                    hv = plsc.load_gather(h_b, [rows, hc + col])
                    rv = plsc.load_gather(r_b, [rows, rc + col])
                    tv = plsc.load_gather(t_b, [rows, tc + col])
                    return acc + jnp.abs(hv + rv - tv)

                acc = lax.fori_loop(
                    0, DIM, d_body, jnp.zeros((LANES,), jnp.float32))
                out_v[pl.ds(j * CHUNK + g * LANES, LANES)] = acc

        issue(0, bufs[0])
        for j in range(n_chunks):
            if j + 1 < n_chunks:
                issue(j + 1, bufs[(j + 1) % 2])
            drain_compute(j, bufs[j % 2])

        pltpu.sync_copy(out_v, out_hbm.at[pl.ds(wid * per_w, per_w)])

    return k


def kernel(positive_triplets, negative_triplets, entities_emb, relations_emb):
    batch = positive_triplets.shape[0]
    total = 2 * batch
    trip = jnp.concatenate(
        [positive_triplets, negative_triplets], axis=0).astype(jnp.int32)
    n_rows = total // CHUNK

    ent_lin = jnp.reshape(entities_emb, (entities_emb.shape[0] // 2, PAD))
    rel_lin = jnp.reshape(relations_emb, (relations_emb.shape[0] // 2, PAD))

    def split(col):
        return ((col // 2).reshape(n_rows, CHUNK),
                (DIM * (col % 2)).reshape(n_rows, CHUNK))

    hrow, hcol = split(trip[:, 0])
    rrow, rcol = split(trip[:, 1])
    trow, tcol = split(trip[:, 2])

    out = _sc_gather(total)(
        hrow, hcol, rrow, rcol, trow, tcol, ent_lin, rel_lin)
    return out[:batch], out[batch:]
